# baseline (speedup 1.0000x reference)
import functools

import jax
import jax.numpy as jnp
from jax import lax
from jax.experimental import pallas as pl
from jax.experimental.pallas import tpu as pltpu
from jax.experimental.pallas import tpu_sc as plsc

BATCH = 4
SEQ_LEN = 8192
D_MODEL = 768
NC, NS, L = 2, 16, 16
NW = NC * NS
ROWS_PER_W = SEQ_LEN // NW
C = 32
NCHUNK = ROWS_PER_W // C
VECS_PER_ROW = D_MODEL // L


def _sc_body(in_hbm, pos_hbm, out_hbm, pos_v, buf_v):
    wid = lax.axis_index("s") * NC + lax.axis_index("c")
    base = wid * ROWS_PER_W

    def chunk_body(ci, _):
        row0 = base + ci * C
        pltpu.sync_copy(pos_hbm.at[pl.ds(row0, C)], pos_v)
        for b in range(BATCH):
            pltpu.sync_copy(in_hbm.at[b, pl.ds(row0, C)], buf_v)

            def add_row(r, _):
                for j in range(VECS_PER_ROW):
                    sl = pl.ds(j * L, L)
                    buf_v[r, sl] = buf_v[r, sl] + pos_v[r, sl]
                return ()

            lax.fori_loop(0, C, add_row, ())
            pltpu.sync_copy(buf_v, out_hbm.at[b, pl.ds(row0, C)])
        return ()

    lax.fori_loop(0, NCHUNK, chunk_body, ())


def kernel(inputs, pos_table):
    mesh = plsc.VectorSubcoreMesh(core_axis_name="c", subcore_axis_name="s")
    run = pl.kernel(
        _sc_body,
        out_type=jax.ShapeDtypeStruct((BATCH, SEQ_LEN, D_MODEL), jnp.float32),
        mesh=mesh,
        scratch_types=[
            pltpu.VMEM((C, D_MODEL), jnp.float32),
            pltpu.VMEM((C, D_MODEL), jnp.float32),
        ],
    )
    return run(inputs, pos_table)


# speedup vs baseline: 2.5517x; 2.5517x over previous
import jax
import jax.numpy as jnp
from jax.experimental import pallas as pl

BATCH = 4
SEQ_LEN = 8192
D_MODEL = 768
BS = 2048


def _add_kernel(x_ref, p_ref, o_ref):
    o_ref[...] = x_ref[...] + p_ref[...]


def kernel(inputs, pos_table):
    grid = (SEQ_LEN // BS, BATCH)
    return pl.pallas_call(
        _add_kernel,
        grid=grid,
        in_specs=[
            pl.BlockSpec((1, BS, D_MODEL), lambda s, b: (b, s, 0)),
            pl.BlockSpec((BS, D_MODEL), lambda s, b: (s, 0)),
        ],
        out_specs=pl.BlockSpec((1, BS, D_MODEL), lambda s, b: (b, s, 0)),
        out_shape=jax.ShapeDtypeStruct((BATCH, SEQ_LEN, D_MODEL), jnp.float32),
    )(inputs, pos_table)
